# baseline (speedup 1.0000x reference)
import functools

import jax
import jax.numpy as jnp
import numpy as np
from jax import lax
from jax.experimental import pallas as pl
from jax.experimental.pallas import tpu as pltpu
from jax.experimental.pallas import tpu_sc as plsc

VOCAB = 100000
EMBED = 64
N_CLASSES = 1000
B = 16384
L = 20

NW = 32
BAGS_PER_W = B // NW
CHUNK_BAGS = 4
CHUNK_IDX = CHUNK_BAGS * L
N_CHUNKS = BAGS_PER_W // CHUNK_BAGS

@functools.cache
def _get_bag_mean_sc():
    mesh = plsc.VectorSubcoreMesh(core_axis_name="c", subcore_axis_name="s")

    @functools.partial(
        pl.kernel,
        out_type=jax.ShapeDtypeStruct((B, EMBED), jnp.float32),
        mesh=mesh,
        scratch_types=[
            pltpu.VMEM((BAGS_PER_W * L,), jnp.int32),
            pltpu.VMEM((CHUNK_IDX, EMBED // 2), jnp.int32),
            pltpu.VMEM((CHUNK_IDX, EMBED // 2), jnp.int32),
            pltpu.VMEM((BAGS_PER_W, EMBED), jnp.float32),
            pltpu.SemaphoreType.DMA,
            pltpu.SemaphoreType.DMA,
        ],
        compiler_params=pltpu.CompilerParams(
            use_tc_tiling_on_sc=False, needs_layout_passes=False
        ),
    )
    def _bag_mean_sc(x_hbm, emb_hbm, out_hbm, idx_v, rows0_v, rows1_v, bag_v,
                     sem0, sem1):
        wid = lax.axis_index("s") * 2 + lax.axis_index("c")
        idx_base = wid * (BAGS_PER_W * L)
        pltpu.sync_copy(x_hbm.at[pl.ds(idx_base, BAGS_PER_W * L)], idx_v)

        bufs = (rows0_v, rows1_v)
        sems = (sem0, sem1)
        himask = jnp.broadcast_to(jnp.int32(-65536), (16,))

        def _gather(ci, buf, sem):
            return pltpu.async_copy(
                emb_hbm.at[idx_v.at[pl.ds(ci * CHUNK_IDX, CHUNK_IDX)]], buf, sem
            )

        _gather(0, bufs[0], sems[0])

        def _compute(ci, buf):
            @pl.loop(0, CHUNK_BAGS)
            def _bag(bi):
                base = bi * L
                accs = [jnp.zeros((16,), jnp.float32) for _ in range(4)]
                for l in range(L):
                    for g in range(2):
                        xi = buf[base + l, pl.ds(16 * g, 16)]
                        even = plsc.bitcast(
                            lax.shift_left(xi, jnp.int32(16)), jnp.float32
                        )
                        odd = plsc.bitcast(xi & himask, jnp.float32)
                        accs[2 * g] = accs[2 * g] + even
                        accs[2 * g + 1] = accs[2 * g + 1] + odd
                out_row = ci * CHUNK_BAGS + bi
                for c in range(4):
                    bag_v[out_row, pl.ds(16 * c, 16)] = accs[c] * (1.0 / L)

        @pl.loop(0, N_CHUNKS // 2)
        def _chunk(ci2):
            for parity in (0, 1):
                ci = ci2 * 2 + parity
                nxt = 1 - parity

                @pl.when(ci + 1 < N_CHUNKS)
                def _():
                    _gather(ci + 1, bufs[nxt], sems[nxt])

                pltpu.make_async_copy(
                    emb_hbm.at[idx_v.at[pl.ds(ci * CHUNK_IDX, CHUNK_IDX)]],
                    bufs[parity],
                    sems[parity],
                ).wait()
                _compute(ci, bufs[parity])

        pltpu.sync_copy(bag_v, out_hbm.at[pl.ds(wid * BAGS_PER_W, BAGS_PER_W)])

    return _bag_mean_sc


_COL_PERM = np.concatenate(
    [np.arange(g * 32, (g + 1) * 32).reshape(16, 2).T.reshape(-1) for g in (0, 1)]
)


MLP_BLK = 2048


def _mlp_t_body(xt_ref, w1t_ref, b1t_ref, w2t_ref, b2t_ref, w3t_ref, b3t_ref,
                outt_ref):
    bf = jnp.bfloat16
    xt = xt_ref[...].astype(bf)
    h = jnp.dot(w1t_ref[...].astype(bf), xt, preferred_element_type=jnp.float32)
    h = jnp.maximum(h + b1t_ref[...], 0.0)
    h = jnp.dot(w2t_ref[...].astype(bf), h.astype(bf),
                preferred_element_type=jnp.float32)
    h = jnp.maximum(h + b2t_ref[...], 0.0)
    outt_ref[...] = (
        jnp.dot(w3t_ref[...].astype(bf), h.astype(bf),
                preferred_element_type=jnp.float32)
        + b3t_ref[...]
    )


def _mlp_t(bag_t, W1t, b1t, W2t, b2t, W3t, b3t):
    return pl.pallas_call(
        _mlp_t_body,
        grid=(B // MLP_BLK,),
        in_specs=[
            pl.BlockSpec((EMBED, MLP_BLK), lambda i: (0, i)),
            pl.BlockSpec((128, EMBED), lambda i: (0, 0)),
            pl.BlockSpec((128, 1), lambda i: (0, 0)),
            pl.BlockSpec((EMBED, 128), lambda i: (0, 0)),
            pl.BlockSpec((EMBED, 1), lambda i: (0, 0)),
            pl.BlockSpec((N_CLASSES, EMBED), lambda i: (0, 0)),
            pl.BlockSpec((N_CLASSES, 1), lambda i: (0, 0)),
        ],
        out_specs=pl.BlockSpec((N_CLASSES, MLP_BLK), lambda i: (0, i)),
        out_shape=jax.ShapeDtypeStruct((N_CLASSES, B), jnp.float32),
    )(bag_t, W1t, b1t, W2t, b2t, W3t, b3t)


def kernel(X_batch, emb, W1, b1, W2, b2, W3, b3):
    x_flat = X_batch.astype(jnp.int32).reshape(-1)
    emb_i32 = jax.lax.bitcast_convert_type(
        emb.astype(jnp.bfloat16).reshape(VOCAB, EMBED // 2, 2), jnp.int32
    )
    bag = _get_bag_mean_sc()(x_flat, emb_i32)
    out_t = _mlp_t(
        bag.T,
        W1[_COL_PERM, :].T,
        b1.reshape(-1, 1),
        W2.T,
        b2.reshape(-1, 1),
        W3.T,
        b3.reshape(-1, 1),
    )
    return out_t.T


# speedup vs baseline: 1.4310x; 1.4310x over previous
import functools

import jax
import jax.numpy as jnp
import numpy as np
from jax import lax
from jax.experimental import pallas as pl
from jax.experimental.pallas import tpu as pltpu
from jax.experimental.pallas import tpu_sc as plsc

VOCAB = 100000
EMBED = 64
N_CLASSES = 1000
B = 16384
L = 20

NW = 32
BAGS_PER_W = B // NW
CHUNK_BAGS = 4
CHUNK_IDX = CHUNK_BAGS * L
N_CHUNKS = BAGS_PER_W // CHUNK_BAGS

@functools.cache
def _get_bag_mean_sc():
    mesh = plsc.VectorSubcoreMesh(core_axis_name="c", subcore_axis_name="s")

    @functools.partial(
        pl.kernel,
        out_type=jax.ShapeDtypeStruct((B, EMBED), jnp.float32),
        mesh=mesh,
        scratch_types=[
            pltpu.VMEM((BAGS_PER_W * L,), jnp.int32),
            pltpu.VMEM((CHUNK_IDX, EMBED // 2), jnp.int32),
            pltpu.VMEM((CHUNK_IDX, EMBED // 2), jnp.int32),
            pltpu.VMEM((BAGS_PER_W, EMBED), jnp.float32),
            pltpu.SemaphoreType.DMA,
            pltpu.SemaphoreType.DMA,
        ],
        compiler_params=pltpu.CompilerParams(
            use_tc_tiling_on_sc=False, needs_layout_passes=False
        ),
    )
    def _bag_mean_sc(x_hbm, emb_hbm, out_hbm, idx_v, rows0_v, rows1_v, bag_v,
                     sem0, sem1):
        wid = lax.axis_index("s") * 2 + lax.axis_index("c")
        idx_base = wid * (BAGS_PER_W * L)
        pltpu.sync_copy(x_hbm.at[pl.ds(idx_base, BAGS_PER_W * L)], idx_v)

        bufs = (rows0_v, rows1_v)
        sems = (sem0, sem1)
        himask = jnp.broadcast_to(jnp.int32(-65536), (16,))

        def _gather(ci, buf, sem):
            return pltpu.async_copy(
                emb_hbm.at[idx_v.at[pl.ds(ci * CHUNK_IDX, CHUNK_IDX)]], buf, sem
            )

        _gather(0, bufs[0], sems[0])

        def _compute(ci, buf):
            @pl.loop(0, CHUNK_BAGS)
            def _bag(bi):
                base = bi * L
                accs = [jnp.zeros((16,), jnp.float32) for _ in range(4)]
                for l in range(L):
                    for g in range(2):
                        xi = buf[base + l, pl.ds(16 * g, 16)]
                        even = plsc.bitcast(
                            lax.shift_left(xi, jnp.int32(16)), jnp.float32
                        )
                        odd = plsc.bitcast(xi & himask, jnp.float32)
                        accs[2 * g] = accs[2 * g] + even
                        accs[2 * g + 1] = accs[2 * g + 1] + odd
                out_row = ci * CHUNK_BAGS + bi
                for c in range(4):
                    bag_v[out_row, pl.ds(16 * c, 16)] = accs[c] * (1.0 / L)

        @pl.loop(0, N_CHUNKS // 2)
        def _chunk(ci2):
            for parity in (0, 1):
                ci = ci2 * 2 + parity
                nxt = 1 - parity

                @pl.when(ci + 1 < N_CHUNKS)
                def _():
                    _gather(ci + 1, bufs[nxt], sems[nxt])

                pltpu.make_async_copy(
                    emb_hbm.at[idx_v.at[pl.ds(ci * CHUNK_IDX, CHUNK_IDX)]],
                    bufs[parity],
                    sems[parity],
                ).wait()
                _compute(ci, bufs[parity])

        pltpu.sync_copy(bag_v, out_hbm.at[pl.ds(wid * BAGS_PER_W, BAGS_PER_W)])

    return _bag_mean_sc


_COL_PERM = np.concatenate(
    [np.arange(0, 16), np.arange(32, 48), np.arange(16, 32), np.arange(48, 64)]
)


MLP_BLK = 2048


def _mlp_t_body(xt_ref, w1t_ref, b1t_ref, w2t_ref, b2t_ref, w3t_ref, b3t_ref,
                outt_ref):
    bf = jnp.bfloat16
    xt = xt_ref[...].astype(bf)
    h = jnp.dot(w1t_ref[...].astype(bf), xt, preferred_element_type=jnp.float32)
    h = jnp.maximum(h + b1t_ref[...], 0.0)
    h = jnp.dot(w2t_ref[...].astype(bf), h.astype(bf),
                preferred_element_type=jnp.float32)
    h = jnp.maximum(h + b2t_ref[...], 0.0)
    outt_ref[...] = (
        jnp.dot(w3t_ref[...].astype(bf), h.astype(bf),
                preferred_element_type=jnp.float32)
        + b3t_ref[...]
    )


def _mlp_t(bag_t, W1t, b1t, W2t, b2t, W3t, b3t):
    return pl.pallas_call(
        _mlp_t_body,
        grid=(B // MLP_BLK,),
        in_specs=[
            pl.BlockSpec((EMBED, MLP_BLK), lambda i: (0, i)),
            pl.BlockSpec((128, EMBED), lambda i: (0, 0)),
            pl.BlockSpec((128, 1), lambda i: (0, 0)),
            pl.BlockSpec((EMBED, 128), lambda i: (0, 0)),
            pl.BlockSpec((EMBED, 1), lambda i: (0, 0)),
            pl.BlockSpec((N_CLASSES, EMBED), lambda i: (0, 0)),
            pl.BlockSpec((N_CLASSES, 1), lambda i: (0, 0)),
        ],
        out_specs=pl.BlockSpec((N_CLASSES, MLP_BLK), lambda i: (0, i)),
        out_shape=jax.ShapeDtypeStruct((N_CLASSES, B), jnp.float32),
    )(bag_t, W1t, b1t, W2t, b2t, W3t, b3t)


def kernel(X_batch, emb, W1, b1, W2, b2, W3, b3):
    x_flat = X_batch.astype(jnp.int32).reshape(-1)
    emb_bits = jax.lax.bitcast_convert_type(
        emb.astype(jnp.bfloat16).astype(jnp.float32), jnp.int32
    )
    emb_i32 = jnp.bitwise_or(
        jax.lax.shift_right_logical(emb_bits[:, : EMBED // 2], 16),
        emb_bits[:, EMBED // 2 :] & jnp.int32(-65536),
    )
    bag = _get_bag_mean_sc()(x_flat, emb_i32)
    out_t = _mlp_t(
        bag.T,
        W1[_COL_PERM, :].T,
        b1.reshape(-1, 1),
        W2.T,
        b2.reshape(-1, 1),
        W3.T,
        b3.reshape(-1, 1),
    )
    return out_t.T


# speedup vs baseline: 2.0780x; 1.4521x over previous
import functools

import jax
import jax.numpy as jnp
import numpy as np
from jax import lax
from jax.experimental import pallas as pl
from jax.experimental.pallas import tpu as pltpu
from jax.experimental.pallas import tpu_sc as plsc

VOCAB = 100000
EMBED = 64
N_CLASSES = 1000
B = 16384
L = 20

NW = 32
BAGS_PER_W = B // NW
CHUNK_BAGS = 4
CHUNK_IDX = CHUNK_BAGS * L
N_CHUNKS = BAGS_PER_W // CHUNK_BAGS

@functools.cache
def _get_bag_mean_sc():
    mesh = plsc.VectorSubcoreMesh(core_axis_name="c", subcore_axis_name="s")

    @functools.partial(
        pl.kernel,
        out_type=jax.ShapeDtypeStruct((B, EMBED), jnp.float32),
        mesh=mesh,
        scratch_types=[
            pltpu.VMEM((BAGS_PER_W * L,), jnp.int32),
            pltpu.VMEM((CHUNK_IDX, EMBED), jnp.float32),
            pltpu.VMEM((CHUNK_IDX, EMBED), jnp.float32),
            pltpu.VMEM((BAGS_PER_W, EMBED), jnp.float32),
            pltpu.SemaphoreType.DMA,
            pltpu.SemaphoreType.DMA,
        ],
        compiler_params=pltpu.CompilerParams(use_tc_tiling_on_sc=False),
    )
    def _bag_mean_sc(x_hbm, emb_hbm, out_hbm, idx_v, rows0_v, rows1_v, bag_v,
                     sem0, sem1):
        wid = lax.axis_index("s") * 2 + lax.axis_index("c")
        idx_base = wid * (BAGS_PER_W * L)
        pltpu.sync_copy(x_hbm.at[pl.ds(idx_base, BAGS_PER_W * L)], idx_v)

        bufs = (rows0_v, rows1_v)
        sems = (sem0, sem1)

        def _gather(ci, buf, sem):
            return pltpu.async_copy(
                emb_hbm.at[idx_v.at[pl.ds(ci * CHUNK_IDX, CHUNK_IDX)]], buf, sem
            )

        _gather(0, bufs[0], sems[0])

        def _compute(ci, buf):
            @pl.loop(0, CHUNK_BAGS)
            def _bag(bi):
                base = bi * L
                accs = [buf[base, pl.ds(16 * c, 16)] for c in range(4)]
                for l in range(1, L):
                    for c in range(4):
                        accs[c] = accs[c] + buf[base + l, pl.ds(16 * c, 16)]
                out_row = ci * CHUNK_BAGS + bi
                for c in range(4):
                    bag_v[out_row, pl.ds(16 * c, 16)] = accs[c] * (1.0 / L)

        @pl.loop(0, N_CHUNKS // 2)
        def _chunk(ci2):
            for parity in (0, 1):
                ci = ci2 * 2 + parity
                nxt = 1 - parity

                @pl.when(ci + 1 < N_CHUNKS)
                def _():
                    _gather(ci + 1, bufs[nxt], sems[nxt])

                pltpu.make_async_copy(
                    emb_hbm.at[idx_v.at[pl.ds(ci * CHUNK_IDX, CHUNK_IDX)]],
                    bufs[parity],
                    sems[parity],
                ).wait()
                _compute(ci, bufs[parity])

        pltpu.sync_copy(bag_v, out_hbm.at[pl.ds(wid * BAGS_PER_W, BAGS_PER_W)])

    return _bag_mean_sc


MLP_BLK = 2048


def _mlp_t_body(x_ref, w1t_ref, b1t_ref, w2t_ref, b2t_ref, w3t_ref, b3t_ref,
                outt_ref):
    bf = jnp.bfloat16
    xt = x_ref[...].astype(bf).T
    h = jnp.dot(w1t_ref[...].astype(bf), xt, preferred_element_type=jnp.float32)
    h = jnp.maximum(h + b1t_ref[...], 0.0)
    h = jnp.dot(w2t_ref[...].astype(bf), h.astype(bf),
                preferred_element_type=jnp.float32)
    h = jnp.maximum(h + b2t_ref[...], 0.0)
    outt_ref[...] = (
        jnp.dot(w3t_ref[...].astype(bf), h.astype(bf),
                preferred_element_type=jnp.float32)
        + b3t_ref[...]
    )


def _mlp_t(bag, W1t, b1t, W2t, b2t, W3t, b3t):
    return pl.pallas_call(
        _mlp_t_body,
        grid=(B // MLP_BLK,),
        in_specs=[
            pl.BlockSpec((MLP_BLK, EMBED), lambda i: (i, 0)),
            pl.BlockSpec((128, EMBED), lambda i: (0, 0)),
            pl.BlockSpec((128, 1), lambda i: (0, 0)),
            pl.BlockSpec((EMBED, 128), lambda i: (0, 0)),
            pl.BlockSpec((EMBED, 1), lambda i: (0, 0)),
            pl.BlockSpec((N_CLASSES, EMBED), lambda i: (0, 0)),
            pl.BlockSpec((N_CLASSES, 1), lambda i: (0, 0)),
        ],
        out_specs=pl.BlockSpec((N_CLASSES, MLP_BLK), lambda i: (0, i)),
        out_shape=jax.ShapeDtypeStruct((N_CLASSES, B), jnp.float32),
    )(bag, W1t, b1t, W2t, b2t, W3t, b3t)


def kernel(X_batch, emb, W1, b1, W2, b2, W3, b3):
    x_flat = X_batch.astype(jnp.int32).reshape(-1) * 2
    emb_pad = jnp.pad(emb, ((0, 0), (0, EMBED))).reshape(2 * VOCAB, EMBED)
    bag = _get_bag_mean_sc()(x_flat, emb_pad)
    out_t = _mlp_t(
        bag,
        W1.T,
        b1.reshape(-1, 1),
        W2.T,
        b2.reshape(-1, 1),
        W3.T,
        b3.reshape(-1, 1),
    )
    return out_t.T


# speedup vs baseline: 2.3265x; 1.1196x over previous
import functools

import jax
import jax.numpy as jnp
import numpy as np
from jax import lax
from jax.experimental import pallas as pl
from jax.experimental.pallas import tpu as pltpu
from jax.experimental.pallas import tpu_sc as plsc

VOCAB = 100000
EMBED = 64
N_CLASSES = 1000
B = 16384
L = 20

NW = 32
BAGS_PER_W = B // NW
CHUNK_BAGS = 8
CHUNK_IDX = CHUNK_BAGS * L
N_CHUNKS = BAGS_PER_W // CHUNK_BAGS

@functools.cache
def _get_bag_mean_sc():
    mesh = plsc.VectorSubcoreMesh(core_axis_name="c", subcore_axis_name="s")

    @functools.partial(
        pl.kernel,
        out_type=jax.ShapeDtypeStruct((B, EMBED), jnp.float32),
        mesh=mesh,
        scratch_types=[
            pltpu.VMEM((BAGS_PER_W * L,), jnp.int32),
            pltpu.VMEM((CHUNK_IDX, EMBED), jnp.float32),
            pltpu.VMEM((CHUNK_IDX, EMBED), jnp.float32),
            pltpu.VMEM((BAGS_PER_W, EMBED), jnp.float32),
            pltpu.SemaphoreType.DMA,
            pltpu.SemaphoreType.DMA,
        ],
        compiler_params=pltpu.CompilerParams(use_tc_tiling_on_sc=False),
    )
    def _bag_mean_sc(x_hbm, emb_hbm, out_hbm, idx_v, rows0_v, rows1_v, bag_v,
                     sem0, sem1):
        wid = lax.axis_index("s") * 2 + lax.axis_index("c")
        idx_base = wid * (BAGS_PER_W * L)
        pltpu.sync_copy(x_hbm.at[pl.ds(idx_base, BAGS_PER_W * L)], idx_v)

        bufs = (rows0_v, rows1_v)
        sems = (sem0, sem1)

        def _gather(ci, buf, sem):
            return pltpu.async_copy(
                emb_hbm.at[idx_v.at[pl.ds(ci * CHUNK_IDX, CHUNK_IDX)]], buf, sem
            )

        _gather(0, bufs[0], sems[0])

        def _compute(ci, buf):
            @pl.loop(0, CHUNK_BAGS)
            def _bag(bi):
                base = bi * L
                accs = [buf[base, pl.ds(16 * c, 16)] for c in range(4)]
                for l in range(1, L):
                    for c in range(4):
                        accs[c] = accs[c] + buf[base + l, pl.ds(16 * c, 16)]
                out_row = ci * CHUNK_BAGS + bi
                for c in range(4):
                    bag_v[out_row, pl.ds(16 * c, 16)] = accs[c] * (1.0 / L)

        @pl.loop(0, N_CHUNKS // 2)
        def _chunk(ci2):
            for parity in (0, 1):
                ci = ci2 * 2 + parity
                nxt = 1 - parity

                @pl.when(ci + 1 < N_CHUNKS)
                def _():
                    _gather(ci + 1, bufs[nxt], sems[nxt])

                pltpu.make_async_copy(
                    emb_hbm.at[idx_v.at[pl.ds(ci * CHUNK_IDX, CHUNK_IDX)]],
                    bufs[parity],
                    sems[parity],
                ).wait()
                _compute(ci, bufs[parity])

        pltpu.sync_copy(bag_v, out_hbm.at[pl.ds(wid * BAGS_PER_W, BAGS_PER_W)])

    return _bag_mean_sc


MLP_BLK = 2048


def _mlp_t_body(x_ref, w1t_ref, b1t_ref, w2t_ref, b2t_ref, w3t_ref, b3t_ref,
                outt_ref):
    bf = jnp.bfloat16
    xt = x_ref[...].astype(bf).T
    h = jnp.dot(w1t_ref[...].astype(bf), xt, preferred_element_type=jnp.float32)
    h = jnp.maximum(h + b1t_ref[...], 0.0)
    h = jnp.dot(w2t_ref[...].astype(bf), h.astype(bf),
                preferred_element_type=jnp.float32)
    h = jnp.maximum(h + b2t_ref[...], 0.0)
    outt_ref[...] = (
        jnp.dot(w3t_ref[...].astype(bf), h.astype(bf),
                preferred_element_type=jnp.float32)
        + b3t_ref[...]
    )


def _mlp_t(bag, W1t, b1t, W2t, b2t, W3t, b3t):
    return pl.pallas_call(
        _mlp_t_body,
        grid=(B // MLP_BLK,),
        in_specs=[
            pl.BlockSpec((MLP_BLK, EMBED), lambda i: (i, 0)),
            pl.BlockSpec((128, EMBED), lambda i: (0, 0)),
            pl.BlockSpec((128, 1), lambda i: (0, 0)),
            pl.BlockSpec((EMBED, 128), lambda i: (0, 0)),
            pl.BlockSpec((EMBED, 1), lambda i: (0, 0)),
            pl.BlockSpec((N_CLASSES, EMBED), lambda i: (0, 0)),
            pl.BlockSpec((N_CLASSES, 1), lambda i: (0, 0)),
        ],
        out_specs=pl.BlockSpec((N_CLASSES, MLP_BLK), lambda i: (0, i)),
        out_shape=jax.ShapeDtypeStruct((N_CLASSES, B), jnp.float32),
    )(bag, W1t, b1t, W2t, b2t, W3t, b3t)


def kernel(X_batch, emb, W1, b1, W2, b2, W3, b3):
    x_flat = X_batch.astype(jnp.int32).reshape(-1) * 2
    emb_pad = jnp.pad(emb, ((0, 0), (0, EMBED))).reshape(2 * VOCAB, EMBED)
    bag = _get_bag_mean_sc()(x_flat, emb_pad)
    out_t = _mlp_t(
        bag,
        W1.T,
        b1.reshape(-1, 1),
        W2.T,
        b2.reshape(-1, 1),
        W3.T,
        b3.reshape(-1, 1),
    )
    return out_t.T


# speedup vs baseline: 2.4791x; 1.0656x over previous
import functools

import jax
import jax.numpy as jnp
import numpy as np
from jax import lax
from jax.experimental import pallas as pl
from jax.experimental.pallas import tpu as pltpu
from jax.experimental.pallas import tpu_sc as plsc

VOCAB = 100000
EMBED = 64
N_CLASSES = 1000
B = 16384
L = 20

NW = 32
BAGS_PER_W = B // NW
CHUNK_BAGS = 16
CHUNK_IDX = CHUNK_BAGS * L
N_CHUNKS = BAGS_PER_W // CHUNK_BAGS

@functools.cache
def _get_bag_mean_sc():
    mesh = plsc.VectorSubcoreMesh(core_axis_name="c", subcore_axis_name="s")

    @functools.partial(
        pl.kernel,
        out_type=jax.ShapeDtypeStruct((B, EMBED), jnp.float32),
        mesh=mesh,
        scratch_types=[
            pltpu.VMEM((BAGS_PER_W * L,), jnp.int32),
            pltpu.VMEM((CHUNK_IDX, EMBED), jnp.float32),
            pltpu.VMEM((CHUNK_IDX, EMBED), jnp.float32),
            pltpu.VMEM((BAGS_PER_W, EMBED), jnp.float32),
            pltpu.SemaphoreType.DMA,
            pltpu.SemaphoreType.DMA,
        ],
        compiler_params=pltpu.CompilerParams(use_tc_tiling_on_sc=False),
    )
    def _bag_mean_sc(x_hbm, emb_hbm, out_hbm, idx_v, rows0_v, rows1_v, bag_v,
                     sem0, sem1):
        wid = lax.axis_index("s") * 2 + lax.axis_index("c")
        idx_base = wid * (BAGS_PER_W * L)
        pltpu.sync_copy(x_hbm.at[pl.ds(idx_base, BAGS_PER_W * L)], idx_v)

        bufs = (rows0_v, rows1_v)
        sems = (sem0, sem1)

        def _gather(ci, buf, sem):
            return pltpu.async_copy(
                emb_hbm.at[idx_v.at[pl.ds(ci * CHUNK_IDX, CHUNK_IDX)]], buf, sem
            )

        _gather(0, bufs[0], sems[0])

        def _compute(ci, buf):
            @pl.loop(0, CHUNK_BAGS)
            def _bag(bi):
                base = bi * L
                accs = [buf[base, pl.ds(16 * c, 16)] for c in range(4)]
                for l in range(1, L):
                    for c in range(4):
                        accs[c] = accs[c] + buf[base + l, pl.ds(16 * c, 16)]
                out_row = ci * CHUNK_BAGS + bi
                for c in range(4):
                    bag_v[out_row, pl.ds(16 * c, 16)] = accs[c] * (1.0 / L)

        @pl.loop(0, N_CHUNKS // 2)
        def _chunk(ci2):
            for parity in (0, 1):
                ci = ci2 * 2 + parity
                nxt = 1 - parity

                @pl.when(ci + 1 < N_CHUNKS)
                def _():
                    _gather(ci + 1, bufs[nxt], sems[nxt])

                pltpu.make_async_copy(
                    emb_hbm.at[idx_v.at[pl.ds(ci * CHUNK_IDX, CHUNK_IDX)]],
                    bufs[parity],
                    sems[parity],
                ).wait()
                _compute(ci, bufs[parity])

        pltpu.sync_copy(bag_v, out_hbm.at[pl.ds(wid * BAGS_PER_W, BAGS_PER_W)])

    return _bag_mean_sc


MLP_BLK = 2048


def _mlp_t_body(x_ref, w1t_ref, b1t_ref, w2t_ref, b2t_ref, w3t_ref, b3t_ref,
                outt_ref):
    bf = jnp.bfloat16
    xt = x_ref[...].astype(bf).T
    h = jnp.dot(w1t_ref[...].astype(bf), xt, preferred_element_type=jnp.float32)
    h = jnp.maximum(h + b1t_ref[...], 0.0)
    h = jnp.dot(w2t_ref[...].astype(bf), h.astype(bf),
                preferred_element_type=jnp.float32)
    h = jnp.maximum(h + b2t_ref[...], 0.0)
    outt_ref[...] = (
        jnp.dot(w3t_ref[...].astype(bf), h.astype(bf),
                preferred_element_type=jnp.float32)
        + b3t_ref[...]
    )


def _mlp_t(bag, W1t, b1t, W2t, b2t, W3t, b3t):
    return pl.pallas_call(
        _mlp_t_body,
        grid=(B // MLP_BLK,),
        in_specs=[
            pl.BlockSpec((MLP_BLK, EMBED), lambda i: (i, 0)),
            pl.BlockSpec((128, EMBED), lambda i: (0, 0)),
            pl.BlockSpec((128, 1), lambda i: (0, 0)),
            pl.BlockSpec((EMBED, 128), lambda i: (0, 0)),
            pl.BlockSpec((EMBED, 1), lambda i: (0, 0)),
            pl.BlockSpec((N_CLASSES, EMBED), lambda i: (0, 0)),
            pl.BlockSpec((N_CLASSES, 1), lambda i: (0, 0)),
        ],
        out_specs=pl.BlockSpec((N_CLASSES, MLP_BLK), lambda i: (0, i)),
        out_shape=jax.ShapeDtypeStruct((N_CLASSES, B), jnp.float32),
    )(bag, W1t, b1t, W2t, b2t, W3t, b3t)


def kernel(X_batch, emb, W1, b1, W2, b2, W3, b3):
    x_flat = X_batch.astype(jnp.int32).reshape(-1) * 2
    emb_pad = jnp.pad(emb, ((0, 0), (0, EMBED))).reshape(2 * VOCAB, EMBED)
    bag = _get_bag_mean_sc()(x_flat, emb_pad)
    out_t = _mlp_t(
        bag,
        W1.T,
        b1.reshape(-1, 1),
        W2.T,
        b2.reshape(-1, 1),
        W3.T,
        b3.reshape(-1, 1),
    )
    return out_t.T


# speedup vs baseline: 2.5357x; 1.0228x over previous
import functools

import jax
import jax.numpy as jnp
import numpy as np
from jax import lax
from jax.experimental import pallas as pl
from jax.experimental.pallas import tpu as pltpu
from jax.experimental.pallas import tpu_sc as plsc

VOCAB = 100000
EMBED = 64
N_CLASSES = 1000
B = 16384
L = 20

NW = 32
BAGS_PER_W = B // NW
CHUNK_BAGS = 32
CHUNK_IDX = CHUNK_BAGS * L
N_CHUNKS = BAGS_PER_W // CHUNK_BAGS

@functools.cache
def _get_bag_mean_sc():
    mesh = plsc.VectorSubcoreMesh(core_axis_name="c", subcore_axis_name="s")

    @functools.partial(
        pl.kernel,
        out_type=jax.ShapeDtypeStruct((B, EMBED), jnp.float32),
        mesh=mesh,
        scratch_types=[
            pltpu.VMEM((BAGS_PER_W * L,), jnp.int32),
            pltpu.VMEM((CHUNK_IDX, EMBED), jnp.float32),
            pltpu.VMEM((CHUNK_IDX, EMBED), jnp.float32),
            pltpu.VMEM((BAGS_PER_W, EMBED), jnp.float32),
            pltpu.SemaphoreType.DMA,
            pltpu.SemaphoreType.DMA,
        ],
        compiler_params=pltpu.CompilerParams(use_tc_tiling_on_sc=False),
    )
    def _bag_mean_sc(x_hbm, emb_hbm, out_hbm, idx_v, rows0_v, rows1_v, bag_v,
                     sem0, sem1):
        wid = lax.axis_index("s") * 2 + lax.axis_index("c")
        idx_base = wid * (BAGS_PER_W * L)
        pltpu.sync_copy(x_hbm.at[pl.ds(idx_base, BAGS_PER_W * L)], idx_v)

        bufs = (rows0_v, rows1_v)
        sems = (sem0, sem1)

        def _gather(ci, buf, sem):
            return pltpu.async_copy(
                emb_hbm.at[idx_v.at[pl.ds(ci * CHUNK_IDX, CHUNK_IDX)]], buf, sem
            )

        _gather(0, bufs[0], sems[0])

        def _compute(ci, buf):
            @pl.loop(0, CHUNK_BAGS)
            def _bag(bi):
                base = bi * L
                accs = [buf[base, pl.ds(16 * c, 16)] for c in range(4)]
                for l in range(1, L):
                    for c in range(4):
                        accs[c] = accs[c] + buf[base + l, pl.ds(16 * c, 16)]
                out_row = ci * CHUNK_BAGS + bi
                for c in range(4):
                    bag_v[out_row, pl.ds(16 * c, 16)] = accs[c] * (1.0 / L)

        @pl.loop(0, N_CHUNKS // 2)
        def _chunk(ci2):
            for parity in (0, 1):
                ci = ci2 * 2 + parity
                nxt = 1 - parity

                @pl.when(ci + 1 < N_CHUNKS)
                def _():
                    _gather(ci + 1, bufs[nxt], sems[nxt])

                pltpu.make_async_copy(
                    emb_hbm.at[idx_v.at[pl.ds(ci * CHUNK_IDX, CHUNK_IDX)]],
                    bufs[parity],
                    sems[parity],
                ).wait()
                _compute(ci, bufs[parity])

        pltpu.sync_copy(bag_v, out_hbm.at[pl.ds(wid * BAGS_PER_W, BAGS_PER_W)])

    return _bag_mean_sc


MLP_BLK = 2048


def _mlp_t_body(x_ref, w1t_ref, b1t_ref, w2t_ref, b2t_ref, w3t_ref, b3t_ref,
                outt_ref):
    bf = jnp.bfloat16
    xt = x_ref[...].astype(bf).T
    h = jnp.dot(w1t_ref[...].astype(bf), xt, preferred_element_type=jnp.float32)
    h = jnp.maximum(h + b1t_ref[...], 0.0)
    h = jnp.dot(w2t_ref[...].astype(bf), h.astype(bf),
                preferred_element_type=jnp.float32)
    h = jnp.maximum(h + b2t_ref[...], 0.0)
    outt_ref[...] = (
        jnp.dot(w3t_ref[...].astype(bf), h.astype(bf),
                preferred_element_type=jnp.float32)
        + b3t_ref[...]
    )


def _mlp_t(bag, W1t, b1t, W2t, b2t, W3t, b3t):
    return pl.pallas_call(
        _mlp_t_body,
        grid=(B // MLP_BLK,),
        in_specs=[
            pl.BlockSpec((MLP_BLK, EMBED), lambda i: (i, 0)),
            pl.BlockSpec((128, EMBED), lambda i: (0, 0)),
            pl.BlockSpec((128, 1), lambda i: (0, 0)),
            pl.BlockSpec((EMBED, 128), lambda i: (0, 0)),
            pl.BlockSpec((EMBED, 1), lambda i: (0, 0)),
            pl.BlockSpec((N_CLASSES, EMBED), lambda i: (0, 0)),
            pl.BlockSpec((N_CLASSES, 1), lambda i: (0, 0)),
        ],
        out_specs=pl.BlockSpec((N_CLASSES, MLP_BLK), lambda i: (0, i)),
        out_shape=jax.ShapeDtypeStruct((N_CLASSES, B), jnp.float32),
    )(bag, W1t, b1t, W2t, b2t, W3t, b3t)


def kernel(X_batch, emb, W1, b1, W2, b2, W3, b3):
    x_flat = X_batch.astype(jnp.int32).reshape(-1) * 2
    emb_pad = jnp.pad(emb, ((0, 0), (0, EMBED))).reshape(2 * VOCAB, EMBED)
    bag = _get_bag_mean_sc()(x_flat, emb_pad)
    out_t = _mlp_t(
        bag,
        W1.T,
        b1.reshape(-1, 1),
        W2.T,
        b2.reshape(-1, 1),
        W3.T,
        b3.reshape(-1, 1),
    )
    return out_t.T


# speedup vs baseline: 2.5425x; 1.0027x over previous
import functools

import jax
import jax.numpy as jnp
from jax import lax
from jax.experimental import pallas as pl
from jax.experimental.pallas import tpu as pltpu
from jax.experimental.pallas import tpu_sc as plsc

VOCAB = 100000
EMBED = 64
N_CLASSES = 1000
B = 16384
L = 20

NW = 32
BAGS_PER_W = B // NW
CHUNK_BAGS = 32
CHUNK_IDX = CHUNK_BAGS * L
N_CHUNKS = BAGS_PER_W // CHUNK_BAGS


@functools.cache
def _get_bag_mean_sc():
    mesh = plsc.VectorSubcoreMesh(core_axis_name="c", subcore_axis_name="s")

    @functools.partial(
        pl.kernel,
        out_type=jax.ShapeDtypeStruct((B, EMBED), jnp.float32),
        mesh=mesh,
        scratch_types=[
            pltpu.VMEM((BAGS_PER_W * L,), jnp.int32),
            pltpu.VMEM((CHUNK_IDX, EMBED), jnp.float32),
            pltpu.VMEM((CHUNK_IDX, EMBED), jnp.float32),
            pltpu.VMEM((BAGS_PER_W, EMBED), jnp.float32),
            pltpu.SemaphoreType.DMA,
            pltpu.SemaphoreType.DMA,
        ],
        compiler_params=pltpu.CompilerParams(use_tc_tiling_on_sc=False),
    )
    def _bag_mean_sc(x_hbm, emb_hbm, out_hbm, idx_v, rows0_v, rows1_v, bag_v,
                     sem0, sem1):
        wid = lax.axis_index("s") * 2 + lax.axis_index("c")
        idx_base = wid * (BAGS_PER_W * L)
        pltpu.sync_copy(x_hbm.at[pl.ds(idx_base, BAGS_PER_W * L)], idx_v)

        bufs = (rows0_v, rows1_v)
        sems = (sem0, sem1)

        def _gather(ci, buf, sem):
            return pltpu.async_copy(
                emb_hbm.at[idx_v.at[pl.ds(ci * CHUNK_IDX, CHUNK_IDX)]], buf, sem
            )

        _gather(0, bufs[0], sems[0])

        def _compute(ci, buf):
            @pl.loop(0, CHUNK_BAGS)
            def _bag(bi):
                base = bi * L
                accs = [buf[base, pl.ds(16 * c, 16)] for c in range(4)]
                for l in range(1, L):
                    for c in range(4):
                        accs[c] = accs[c] + buf[base + l, pl.ds(16 * c, 16)]
                out_row = ci * CHUNK_BAGS + bi
                for c in range(4):
                    bag_v[out_row, pl.ds(16 * c, 16)] = accs[c] * (1.0 / L)

        @pl.loop(0, N_CHUNKS // 2)
        def _chunk(ci2):
            for parity in (0, 1):
                ci = ci2 * 2 + parity
                nxt = 1 - parity

                @pl.when(ci + 1 < N_CHUNKS)
                def _():
                    _gather(ci + 1, bufs[nxt], sems[nxt])

                pltpu.make_async_copy(
                    emb_hbm.at[idx_v.at[pl.ds(ci * CHUNK_IDX, CHUNK_IDX)]],
                    bufs[parity],
                    sems[parity],
                ).wait()
                _compute(ci, bufs[parity])

        pltpu.sync_copy(bag_v, out_hbm.at[pl.ds(wid * BAGS_PER_W, BAGS_PER_W)])

    return _bag_mean_sc


MLP_BLK = 2048


def _mlp_t_body(x_ref, w1t_ref, b1t_ref, w2t_ref, b2t_ref, w3t_ref, b3t_ref,
                outt_ref):
    bf = jnp.bfloat16
    xt = x_ref[...].astype(bf).T
    h = jnp.dot(w1t_ref[...].astype(bf), xt, preferred_element_type=jnp.float32)
    h = jnp.maximum(h + b1t_ref[...], 0.0)
    h = jnp.dot(w2t_ref[...].astype(bf), h.astype(bf),
                preferred_element_type=jnp.float32)
    h = jnp.maximum(h + b2t_ref[...], 0.0)
    outt_ref[...] = (
        jnp.dot(w3t_ref[...].astype(bf), h.astype(bf),
                preferred_element_type=jnp.float32)
        + b3t_ref[...]
    )


def _mlp_t(bag, W1t, b1t, W2t, b2t, W3t, b3t):
    return pl.pallas_call(
        _mlp_t_body,
        grid=(B // MLP_BLK,),
        in_specs=[
            pl.BlockSpec((MLP_BLK, EMBED), lambda i: (i, 0)),
            pl.BlockSpec((128, EMBED), lambda i: (0, 0)),
            pl.BlockSpec((128, 1), lambda i: (0, 0)),
            pl.BlockSpec((EMBED, 128), lambda i: (0, 0)),
            pl.BlockSpec((EMBED, 1), lambda i: (0, 0)),
            pl.BlockSpec((N_CLASSES, EMBED), lambda i: (0, 0)),
            pl.BlockSpec((N_CLASSES, 1), lambda i: (0, 0)),
        ],
        out_specs=pl.BlockSpec((N_CLASSES, MLP_BLK), lambda i: (0, i)),
        out_shape=jax.ShapeDtypeStruct((N_CLASSES, B), jnp.float32),
    )(bag, W1t, b1t, W2t, b2t, W3t, b3t)


def kernel(X_batch, emb, W1, b1, W2, b2, W3, b3):
    x_flat = X_batch.astype(jnp.int32).reshape(-1) * 2
    emb_pad = jnp.pad(emb, ((0, 0), (0, EMBED))).reshape(2 * VOCAB, EMBED)
    bag = _get_bag_mean_sc()(x_flat, emb_pad)
    out_t = _mlp_t(
        bag,
        W1.T,
        b1.reshape(-1, 1),
        W2.T,
        b2.reshape(-1, 1),
        W3.T,
        b3.reshape(-1, 1),
    )
    return out_t.T
